# Initial kernel scaffold; baseline (speedup 1.0000x reference)
import jax
import jax.numpy as jnp
from jax.experimental import pallas as pl


def kernel(feats, coors, W_qkv, W_out, b_out, W_c1, b_c1, W_c2, b_c2, ln_w, ln_b):
    raise NotImplementedError("write your pallas kernel here")


# speedup vs baseline: 1.9914x; 1.9914x over previous
import jax
import jax.numpy as jnp
from jax.lax import Precision as _P
from jax.experimental import pallas as pl

B, N, DIM = 2, 1024, 512
HEADS, DIM_HEAD, M_DIM, NEIGHBORS = 8, 64, 4, 32
INNER = HEADS * DIM_HEAD
SCALE = DIM_HEAD ** -0.5
ROT_DIM = DIM_HEAD // 2

MB = 256
RB = 256
NB = 16
NBK = NB * NEIGHBORS


def _proj_body(x_ref, wq_ref, wkv_ref, q_ref, kv_ref):
    x = x_ref[...]
    q_ref[...] = jnp.dot(x, wq_ref[...], preferred_element_type=jnp.float32, precision=_P.HIGHEST)
    kv_ref[...] = jnp.dot(x, wkv_ref[...], preferred_element_type=jnp.float32, precision=_P.HIGHEST)


def _topk_body(crow_ref, ct_ref, idx_ref):
    cr = crow_ref[0]
    ca = ct_ref[0]
    dx = cr[:, 0:1] - ca[0:1, :]
    dy = cr[:, 1:2] - ca[1:2, :]
    dz = cr[:, 2:3] - ca[2:3, :]
    cur = dx * dx + dy * dy + dz * dz
    iota = jax.lax.broadcasted_iota(jnp.int32, (RB, N), 1)
    cols = []
    for _ in range(NEIGHBORS):
        m = jnp.min(cur, axis=1, keepdims=True)
        cand = jnp.where(cur == m, iota, N)
        amin = jnp.min(cand, axis=1, keepdims=True)
        cols.append(amin)
        cur = jnp.where(iota == amin, jnp.inf, cur)
    idx_ref[...] = jnp.concatenate(cols, axis=1)[None]


def _attn_body(q_ref, kv_ref, ca_ref, cr_ref, idx_ref, hsum_ref, hexp_ref,
               iffr_ref, wc1_ref, bc1_ref, wc2_ref, bc2_ref, lnb_ref,
               wout_ref, bout_ref, out_ref, cout_ref):
    idxc = idx_ref[0]
    iota = jax.lax.broadcasted_iota(jnp.int32, (NBK, N), 1)
    onehot = (idxc == iota).astype(jnp.float32)
    kv_sel = jnp.dot(onehot, kv_ref[0], preferred_element_type=jnp.float32)
    c_sel = jnp.dot(onehot, ca_ref[0], preferred_element_type=jnp.float32, precision=_P.HIGHEST)
    cr = cr_ref[0]
    c_ctr = jnp.broadcast_to(cr[:, None, :], (NB, NEIGHBORS, 3)).reshape(NBK, 3)
    rel = c_ctr - c_sel
    norm = jnp.sqrt(jnp.sum(rel * rel, axis=1, keepdims=True) + 1e-12)

    theta = norm * iffr_ref[...]
    cth = jnp.cos(theta)
    sth = jnp.sin(theta)
    k_sel = kv_sel[:, :INNER]
    v_sel = kv_sel[:, INNER:]
    lane = jax.lax.broadcasted_iota(jnp.int32, (1, INNER), 1)
    even = (lane % 2) == 0

    def rot(x):
        rl = jnp.concatenate([x[:, 1:], x[:, :1]], axis=1)
        rr = jnp.concatenate([x[:, -1:], x[:, :-1]], axis=1)
        return jnp.where(even, -rl, rr)

    k_rot = k_sel * cth + rot(k_sel) * sth
    v_rot = v_sel * cth + rot(v_sel) * sth

    q = q_ref[0]
    q_rep = jnp.broadcast_to(q[:, None, :], (NB, NEIGHBORS, INNER)).reshape(NBK, INNER)
    qk2 = jnp.dot(q_rep * k_rot, hsum_ref[...],
                  preferred_element_type=jnp.float32, precision=_P.HIGHEST) * SCALE

    h = jnp.dot(qk2, wc1_ref[...], preferred_element_type=jnp.float32, precision=_P.HIGHEST) + bc1_ref[...]
    h = 0.5 * h * (1.0 + jax.lax.erf(h * (2.0 ** -0.5)))
    cw = jnp.dot(h, wc2_ref[...], preferred_element_type=jnp.float32, precision=_P.HIGHEST) + bc2_ref[...]

    normed = rel / jnp.maximum(norm, 1e-8)
    reln = lnb_ref[0, 0] * normed
    wrel = cw * reln
    cout_ref[...] = jnp.sum(wrel.reshape(NB, NEIGHBORS, 3), axis=1)[None]

    qk3 = qk2.reshape(NB, NEIGHBORS, HEADS)
    mx = jnp.max(qk3, axis=1, keepdims=True)
    e = jnp.exp(qk3 - mx)
    attn = e / jnp.sum(e, axis=1, keepdims=True)
    aexp = jnp.dot(attn.reshape(NBK, HEADS), hexp_ref[...],
                   preferred_element_type=jnp.float32, precision=_P.HIGHEST)
    osum = jnp.sum((aexp * v_rot).reshape(NB, NEIGHBORS, INNER), axis=1)
    out_ref[...] = (jnp.dot(osum, wout_ref[...],
                            preferred_element_type=jnp.float32, precision=_P.HIGHEST) + bout_ref[...])[None]


def kernel(feats, coors, W_qkv, W_out, b_out, W_c1, b_c1, W_c2, b_c2, ln_w, ln_b):
    f32 = jnp.float32
    x = feats.reshape(B * N, DIM)
    Wq = W_qkv[:, :INNER]
    Wkv = W_qkv[:, INNER:]
    q2, kv2 = pl.pallas_call(
        _proj_body,
        grid=(B * N // MB,),
        in_specs=[
            pl.BlockSpec((MB, DIM), lambda i: (i, 0)),
            pl.BlockSpec((DIM, INNER), lambda i: (0, 0)),
            pl.BlockSpec((DIM, 2 * INNER), lambda i: (0, 0)),
        ],
        out_specs=[
            pl.BlockSpec((MB, INNER), lambda i: (i, 0)),
            pl.BlockSpec((MB, 2 * INNER), lambda i: (i, 0)),
        ],
        out_shape=[
            jax.ShapeDtypeStruct((B * N, INNER), f32),
            jax.ShapeDtypeStruct((B * N, 2 * INNER), f32),
        ],
    )(x, Wq, Wkv)
    q = q2.reshape(B, N, INNER)
    kv = kv2.reshape(B, N, 2 * INNER)

    coorsT = jnp.transpose(coors, (0, 2, 1))
    idx = pl.pallas_call(
        _topk_body,
        grid=(B, N // RB),
        in_specs=[
            pl.BlockSpec((1, RB, 3), lambda b, r: (b, r, 0)),
            pl.BlockSpec((1, 3, N), lambda b, r: (b, 0, 0)),
        ],
        out_specs=pl.BlockSpec((1, RB, NEIGHBORS), lambda b, r: (b, r, 0)),
        out_shape=jax.ShapeDtypeStruct((B, N, NEIGHBORS), jnp.int32),
    )(coors, coorsT)
    idxf = idx.reshape(B, N * NEIGHBORS, 1)

    dh = jnp.arange(INNER, dtype=jnp.int32) // DIM_HEAD
    hsum = (dh[:, None] == jnp.arange(HEADS, dtype=jnp.int32)[None, :]).astype(f32)
    hexp = hsum.T
    r = jnp.arange(INNER, dtype=jnp.int32) % DIM_HEAD
    inv_freq = 1.0 / (10000.0 ** (jnp.arange(0, ROT_DIM, 2, dtype=f32) / ROT_DIM))
    iffr = jnp.where(r < ROT_DIM,
                     100.0 * inv_freq[jnp.minimum(r, ROT_DIM - 1) // 2],
                     0.0)[None, :].astype(f32)

    out, coors_out = pl.pallas_call(
        _attn_body,
        grid=(B, N // NB),
        in_specs=[
            pl.BlockSpec((1, NB, INNER), lambda b, i: (b, i, 0)),
            pl.BlockSpec((1, N, 2 * INNER), lambda b, i: (b, 0, 0)),
            pl.BlockSpec((1, N, 3), lambda b, i: (b, 0, 0)),
            pl.BlockSpec((1, NB, 3), lambda b, i: (b, i, 0)),
            pl.BlockSpec((1, NBK, 1), lambda b, i: (b, i, 0)),
            pl.BlockSpec((INNER, HEADS), lambda b, i: (0, 0)),
            pl.BlockSpec((HEADS, INNER), lambda b, i: (0, 0)),
            pl.BlockSpec((1, INNER), lambda b, i: (0, 0)),
            pl.BlockSpec((HEADS, M_DIM * 4), lambda b, i: (0, 0)),
            pl.BlockSpec((1, M_DIM * 4), lambda b, i: (0, 0)),
            pl.BlockSpec((M_DIM * 4, 1), lambda b, i: (0, 0)),
            pl.BlockSpec((1, 1), lambda b, i: (0, 0)),
            pl.BlockSpec((1, 1), lambda b, i: (0, 0)),
            pl.BlockSpec((INNER, DIM), lambda b, i: (0, 0)),
            pl.BlockSpec((1, DIM), lambda b, i: (0, 0)),
        ],
        out_specs=[
            pl.BlockSpec((1, NB, DIM), lambda b, i: (b, i, 0)),
            pl.BlockSpec((1, NB, 3), lambda b, i: (b, i, 0)),
        ],
        out_shape=[
            jax.ShapeDtypeStruct((B, N, DIM), f32),
            jax.ShapeDtypeStruct((B, N, 3), f32),
        ],
    )(q, kv, coors, coors, idxf, hsum, hexp, iffr, W_c1,
      b_c1.reshape(1, -1), W_c2,
      (b_c2 + 0.0).reshape(1, 1), ln_b.reshape(1, 1), W_out,
      b_out.reshape(1, -1))
    return out, coors_out


# speedup vs baseline: 2.7575x; 1.3847x over previous
import jax
import jax.numpy as jnp
from jax.lax import Precision as _P
from jax.experimental import pallas as pl

B, N, DIM = 2, 1024, 512
HEADS, DIM_HEAD, M_DIM, NEIGHBORS = 8, 64, 4, 32
INNER = HEADS * DIM_HEAD
SCALE = DIM_HEAD ** -0.5
ROT_DIM = DIM_HEAD // 2
NFREQ = ROT_DIM // 2
KVC = 2 * INNER + 128

MB = 256
RB = 256
NB = 16
NBK = NB * NEIGHBORS


def _proj_body(x_ref, c_ref, wq_ref, wkv_ref, q_ref, kvc_ref):
    x = x_ref[...]
    q_ref[...] = jnp.dot(x, wq_ref[...], preferred_element_type=jnp.float32,
                         precision=_P.HIGHEST)
    kvc_ref[:, :2 * INNER] = jnp.dot(x, wkv_ref[...],
                                     preferred_element_type=jnp.float32,
                                     precision=_P.HIGHEST)
    c = c_ref[...]
    chi = c.astype(jnp.bfloat16).astype(jnp.float32)
    clo = c - chi
    kvc_ref[:, 2 * INNER:] = jnp.concatenate(
        [chi, clo, jnp.zeros((MB, 122), jnp.float32)], axis=1)


def _topk_body(crow_ref, ct_ref, idx_ref):
    cr = crow_ref[0]
    ca = ct_ref[0]
    dx = cr[:, 0:1] - ca[0:1, :]
    dy = cr[:, 1:2] - ca[1:2, :]
    dz = cr[:, 2:3] - ca[2:3, :]
    cur = dx * dx + dy * dy + dz * dz
    iota = jax.lax.broadcasted_iota(jnp.int32, (RB, N), 1)
    cols = []
    for _ in range(NEIGHBORS):
        m = jnp.min(cur, axis=1, keepdims=True)
        cand = jnp.where(cur == m, iota, N)
        amin = jnp.min(cand, axis=1, keepdims=True)
        cols.append(amin)
        cur = jnp.where(iota == amin, jnp.inf, cur)
    idx_ref[...] = jnp.concatenate(cols, axis=1)[None]


def _attn_body(q_ref, kvc_ref, cr_ref, idx_ref, hsum_ref, hexp_ref,
               iffr_ref, emat_ref, pmask_ref, wc1_ref, bc1_ref, wc2_ref,
               bc2_ref, lnb_ref, wout_ref, bout_ref, out_ref, cout_ref):
    idxc = idx_ref[0]
    iota = jax.lax.broadcasted_iota(jnp.int32, (NBK, N), 1)
    onehot = (idxc == iota).astype(jnp.float32)
    sel = jnp.dot(onehot, kvc_ref[0], preferred_element_type=jnp.float32)
    k_sel = sel[:, :INNER]
    v_sel = sel[:, INNER:2 * INNER]
    c_sel = sel[:, 2 * INNER:2 * INNER + 3] + sel[:, 2 * INNER + 3:2 * INNER + 6]
    cr = cr_ref[0]
    c_ctr = jnp.broadcast_to(cr[:, None, :], (NB, NEIGHBORS, 3)).reshape(NBK, 3)
    rel = c_ctr - c_sel
    norm = jnp.sqrt(jnp.sum(rel * rel, axis=1, keepdims=True) + 1e-12)

    th16 = norm * iffr_ref[...]
    c16 = jnp.cos(th16)
    s16 = jnp.sin(th16)
    emat = emat_ref[...]
    cth = jnp.dot(c16, emat, preferred_element_type=jnp.float32,
                  precision=_P.HIGHEST) + pmask_ref[...]
    sth = jnp.dot(s16, emat, preferred_element_type=jnp.float32,
                  precision=_P.HIGHEST)
    lane = jax.lax.broadcasted_iota(jnp.int32, (1, INNER), 1)
    even = (lane % 2) == 0

    def rot(x):
        rl = jnp.concatenate([x[:, 1:], x[:, :1]], axis=1)
        rr = jnp.concatenate([x[:, -1:], x[:, :-1]], axis=1)
        return jnp.where(even, -rl, rr)

    k_rot = k_sel * cth + rot(k_sel) * sth
    v_rot = v_sel * cth + rot(v_sel) * sth

    q = q_ref[0]
    q_rep = jnp.broadcast_to(q[:, None, :], (NB, NEIGHBORS, INNER)).reshape(NBK, INNER)
    qk2 = jnp.dot(q_rep * k_rot, hsum_ref[...],
                  preferred_element_type=jnp.float32,
                  precision=_P.HIGHEST) * SCALE

    h = jnp.dot(qk2, wc1_ref[...], preferred_element_type=jnp.float32,
                precision=_P.HIGHEST) + bc1_ref[...]
    h = 0.5 * h * (1.0 + jax.lax.erf(h * (2.0 ** -0.5)))
    cw = jnp.dot(h, wc2_ref[...], preferred_element_type=jnp.float32,
                 precision=_P.HIGHEST) + bc2_ref[...]

    normed = rel / jnp.maximum(norm, 1e-8)
    reln = lnb_ref[0, 0] * normed
    wrel = cw * reln
    cout_ref[...] = jnp.sum(wrel.reshape(NB, NEIGHBORS, 3), axis=1)[None]

    qk3 = qk2.reshape(NB, NEIGHBORS, HEADS)
    mx = jnp.max(qk3, axis=1, keepdims=True)
    e = jnp.exp(qk3 - mx)
    attn = e / jnp.sum(e, axis=1, keepdims=True)
    aexp = jnp.dot(attn.reshape(NBK, HEADS), hexp_ref[...],
                   preferred_element_type=jnp.float32,
                   precision=_P.HIGHEST)
    osum = jnp.sum((aexp * v_rot).reshape(NB, NEIGHBORS, INNER), axis=1)
    out_ref[...] = (jnp.dot(osum, wout_ref[...],
                            preferred_element_type=jnp.float32,
                            precision=_P.HIGHEST) + bout_ref[...])[None]


def kernel(feats, coors, W_qkv, W_out, b_out, W_c1, b_c1, W_c2, b_c2, ln_w, ln_b):
    f32 = jnp.float32
    x = feats.reshape(B * N, DIM)
    cflat = coors.reshape(B * N, 3)
    Wq = W_qkv[:, :INNER]
    Wkv = W_qkv[:, INNER:]
    q2, kvc = pl.pallas_call(
        _proj_body,
        grid=(B * N // MB,),
        in_specs=[
            pl.BlockSpec((MB, DIM), lambda i: (i, 0)),
            pl.BlockSpec((MB, 3), lambda i: (i, 0)),
            pl.BlockSpec((DIM, INNER), lambda i: (0, 0)),
            pl.BlockSpec((DIM, 2 * INNER), lambda i: (0, 0)),
        ],
        out_specs=[
            pl.BlockSpec((MB, INNER), lambda i: (i, 0)),
            pl.BlockSpec((MB, KVC), lambda i: (i, 0)),
        ],
        out_shape=[
            jax.ShapeDtypeStruct((B * N, INNER), f32),
            jax.ShapeDtypeStruct((B * N, KVC), f32),
        ],
    )(x, cflat, Wq, Wkv)
    q = q2.reshape(B, N, INNER)
    kvc = kvc.reshape(B, N, KVC)

    coorsT = jnp.transpose(coors, (0, 2, 1))
    idx = pl.pallas_call(
        _topk_body,
        grid=(B, N // RB),
        in_specs=[
            pl.BlockSpec((1, RB, 3), lambda b, r: (b, r, 0)),
            pl.BlockSpec((1, 3, N), lambda b, r: (b, 0, 0)),
        ],
        out_specs=pl.BlockSpec((1, RB, NEIGHBORS), lambda b, r: (b, r, 0)),
        out_shape=jax.ShapeDtypeStruct((B, N, NEIGHBORS), jnp.int32),
    )(coors, coorsT)
    idxf = idx.reshape(B, N * NEIGHBORS, 1)

    dh = jnp.arange(INNER, dtype=jnp.int32) // DIM_HEAD
    hsum = (dh[:, None] == jnp.arange(HEADS, dtype=jnp.int32)[None, :]).astype(f32)
    hexp = hsum.T
    dm = jnp.arange(INNER, dtype=jnp.int32) % DIM_HEAD
    inv_freq = 1.0 / (10000.0 ** (jnp.arange(0, ROT_DIM, 2, dtype=f32) / ROT_DIM))
    iffr16 = (100.0 * inv_freq)[None, :]
    emat = ((dm[None, :] < ROT_DIM)
            & ((dm[None, :] // 2) == jnp.arange(NFREQ, dtype=jnp.int32)[:, None])
            ).astype(f32)
    pmask = (dm >= ROT_DIM).astype(f32)[None, :]

    out, coors_out = pl.pallas_call(
        _attn_body,
        grid=(B, N // NB),
        in_specs=[
            pl.BlockSpec((1, NB, INNER), lambda b, i: (b, i, 0)),
            pl.BlockSpec((1, N, KVC), lambda b, i: (b, 0, 0)),
            pl.BlockSpec((1, NB, 3), lambda b, i: (b, i, 0)),
            pl.BlockSpec((1, NBK, 1), lambda b, i: (b, i, 0)),
            pl.BlockSpec((INNER, HEADS), lambda b, i: (0, 0)),
            pl.BlockSpec((HEADS, INNER), lambda b, i: (0, 0)),
            pl.BlockSpec((1, NFREQ), lambda b, i: (0, 0)),
            pl.BlockSpec((NFREQ, INNER), lambda b, i: (0, 0)),
            pl.BlockSpec((1, INNER), lambda b, i: (0, 0)),
            pl.BlockSpec((HEADS, M_DIM * 4), lambda b, i: (0, 0)),
            pl.BlockSpec((1, M_DIM * 4), lambda b, i: (0, 0)),
            pl.BlockSpec((M_DIM * 4, 1), lambda b, i: (0, 0)),
            pl.BlockSpec((1, 1), lambda b, i: (0, 0)),
            pl.BlockSpec((1, 1), lambda b, i: (0, 0)),
            pl.BlockSpec((INNER, DIM), lambda b, i: (0, 0)),
            pl.BlockSpec((1, DIM), lambda b, i: (0, 0)),
        ],
        out_specs=[
            pl.BlockSpec((1, NB, DIM), lambda b, i: (b, i, 0)),
            pl.BlockSpec((1, NB, 3), lambda b, i: (b, i, 0)),
        ],
        out_shape=[
            jax.ShapeDtypeStruct((B, N, DIM), f32),
            jax.ShapeDtypeStruct((B, N, 3), f32),
        ],
    )(q, kvc, coors, idxf, hsum, hexp, iffr16, emat, pmask, W_c1,
      b_c1.reshape(1, -1), W_c2, (b_c2 + 0.0).reshape(1, 1),
      ln_b.reshape(1, 1), W_out, b_out.reshape(1, -1))
    return out, coors_out


# speedup vs baseline: 5.0002x; 1.8133x over previous
import jax
import jax.numpy as jnp
from jax.experimental import pallas as pl

B, N, DIM = 2, 1024, 512
HEADS, DIM_HEAD, M_DIM, NEIGHBORS = 8, 64, 4, 32
INNER = HEADS * DIM_HEAD
SCALE = DIM_HEAD ** -0.5
ROT_DIM = DIM_HEAD // 2
NFREQ = ROT_DIM // 2
KVC = 2 * INNER + 256

MB = 256
RB = 256
NB = 16
NBK = NB * NEIGHBORS


def _proj_body(x_ref, c_ref, wq_ref, wkv_ref, q_ref, kvc_ref):
    x = x_ref[...]
    q_ref[...] = jnp.dot(x, wq_ref[...], preferred_element_type=jnp.float32)
    kvc_ref[:, :2 * INNER] = jnp.dot(x, wkv_ref[...],
                                     preferred_element_type=jnp.float32)
    c = c_ref[...]
    chi = c.astype(jnp.bfloat16).astype(jnp.float32)
    clo = c - chi
    z = jnp.zeros((MB, 125), jnp.float32)
    kvc_ref[:, 2 * INNER:] = jnp.concatenate([chi, z, clo, z], axis=1)


def _topk_body(crow_ref, ct_ref, idx_ref):
    cr = crow_ref[0]
    ca = ct_ref[0]
    dx = cr[:, 0:1] - ca[0:1, :]
    dy = cr[:, 1:2] - ca[1:2, :]
    dz = cr[:, 2:3] - ca[2:3, :]
    cur = dx * dx + dy * dy + dz * dz
    iota = jax.lax.broadcasted_iota(jnp.int32, (RB, N), 1)
    lane32 = jax.lax.broadcasted_iota(jnp.int32, (RB, NEIGHBORS), 1)
    acc = jnp.zeros((RB, NEIGHBORS), jnp.int32)
    for t in range(NEIGHBORS):
        m = jnp.min(cur, axis=1, keepdims=True)
        cand = jnp.where(cur == m, iota, N)
        amin = jnp.min(cand, axis=1, keepdims=True)
        acc = jnp.where(lane32 == t, amin, acc)
        cur = jnp.where(iota == amin, jnp.inf, cur)
    idx_ref[...] = acc[None]


def _attn_body(q_ref, kvc_ref, cr_ref, idx_ref, hsum_ref, hexp_ref,
               iffr_ref, emat_ref, pmask_ref, wc1_ref, bc1_ref, wc2_ref,
               bc2_ref, lnb_ref, wout_ref, bout_ref, out_ref, cout_ref):
    idxc = idx_ref[0]
    iota = jax.lax.broadcasted_iota(jnp.int32, (NBK, N), 1)
    onehot = (idxc == iota).astype(jnp.float32)
    sel = jnp.dot(onehot, kvc_ref[0], preferred_element_type=jnp.float32)
    k_sel = sel[:, :INNER]
    v_sel = sel[:, INNER:2 * INNER]
    c_sel = (sel[:, 2 * INNER:2 * INNER + 3]
             + sel[:, 2 * INNER + 128:2 * INNER + 131])
    cr = cr_ref[0]
    c_ctr = jnp.broadcast_to(cr[:, None, :], (NB, NEIGHBORS, 3)).reshape(NBK, 3)
    rel = c_ctr - c_sel
    norm = jnp.sqrt(jnp.sum(rel * rel, axis=1, keepdims=True) + 1e-12)

    th16 = norm * iffr_ref[...]
    c16 = jnp.cos(th16)
    s16 = jnp.sin(th16)
    c16h = c16.astype(jnp.bfloat16).astype(jnp.float32)
    s16h = s16.astype(jnp.bfloat16).astype(jnp.float32)
    csin = jnp.concatenate([c16h, c16 - c16h, s16h, s16 - s16h], axis=1)
    cs = jnp.dot(csin, emat_ref[...], preferred_element_type=jnp.float32)
    cth = cs[:, :INNER] + pmask_ref[...]
    sth = cs[:, INNER:]
    lane = jax.lax.broadcasted_iota(jnp.int32, (1, INNER), 1)
    even = (lane % 2) == 0

    def rot(x):
        rl = jnp.concatenate([x[:, 1:], x[:, :1]], axis=1)
        rr = jnp.concatenate([x[:, -1:], x[:, :-1]], axis=1)
        return jnp.where(even, -rl, rr)

    k_rot = k_sel * cth + rot(k_sel) * sth
    v_rot = v_sel * cth + rot(v_sel) * sth

    q = q_ref[0]
    q_rep = jnp.broadcast_to(q[:, None, :], (NB, NEIGHBORS, INNER)).reshape(NBK, INNER)
    qk2 = jnp.dot(q_rep * k_rot, hsum_ref[...],
                  preferred_element_type=jnp.float32) * SCALE

    h = jnp.dot(qk2, wc1_ref[...], preferred_element_type=jnp.float32) + bc1_ref[...]
    h = 0.5 * h * (1.0 + jax.lax.erf(h * (2.0 ** -0.5)))
    cw = jnp.dot(h, wc2_ref[...], preferred_element_type=jnp.float32) + bc2_ref[...]

    normed = rel / jnp.maximum(norm, 1e-8)
    reln = lnb_ref[0, 0] * normed
    wrel = cw * reln
    cout_ref[...] = jnp.sum(wrel.reshape(NB, NEIGHBORS, 3), axis=1)[None]

    qk3 = qk2.reshape(NB, NEIGHBORS, HEADS)
    mx = jnp.max(qk3, axis=1, keepdims=True)
    e = jnp.exp(qk3 - mx)
    attn = e / jnp.sum(e, axis=1, keepdims=True)
    aexp = jnp.dot(attn.reshape(NBK, HEADS), hexp_ref[...],
                   preferred_element_type=jnp.float32)
    osum = jnp.sum((aexp * v_rot).reshape(NB, NEIGHBORS, INNER), axis=1)
    osum2 = jnp.concatenate([osum, osum], axis=1)
    out_ref[...] = (jnp.dot(osum2, wout_ref[...],
                            preferred_element_type=jnp.float32)
                    + bout_ref[...])[None]


def kernel(feats, coors, W_qkv, W_out, b_out, W_c1, b_c1, W_c2, b_c2, ln_w, ln_b):
    f32 = jnp.float32
    x = feats.reshape(B * N, DIM)
    cflat = coors.reshape(B * N, 3)
    Wq = W_qkv[:, :INNER]
    Wkv = W_qkv[:, INNER:]
    q2, kvc = pl.pallas_call(
        _proj_body,
        grid=(B * N // MB,),
        in_specs=[
            pl.BlockSpec((MB, DIM), lambda i: (i, 0)),
            pl.BlockSpec((MB, 3), lambda i: (i, 0)),
            pl.BlockSpec((DIM, INNER), lambda i: (0, 0)),
            pl.BlockSpec((DIM, 2 * INNER), lambda i: (0, 0)),
        ],
        out_specs=[
            pl.BlockSpec((MB, INNER), lambda i: (i, 0)),
            pl.BlockSpec((MB, KVC), lambda i: (i, 0)),
        ],
        out_shape=[
            jax.ShapeDtypeStruct((B * N, INNER), f32),
            jax.ShapeDtypeStruct((B * N, KVC), f32),
        ],
    )(x, cflat, Wq, Wkv)
    q = q2.reshape(B, N, INNER)
    kvc = kvc.reshape(B, N, KVC)

    coorsT = jnp.transpose(coors, (0, 2, 1))
    idx = pl.pallas_call(
        _topk_body,
        grid=(B, N // RB),
        in_specs=[
            pl.BlockSpec((1, RB, 3), lambda b, r: (b, r, 0)),
            pl.BlockSpec((1, 3, N), lambda b, r: (b, 0, 0)),
        ],
        out_specs=pl.BlockSpec((1, RB, NEIGHBORS), lambda b, r: (b, r, 0)),
        out_shape=jax.ShapeDtypeStruct((B, N, NEIGHBORS), jnp.int32),
    )(coors, coorsT)
    idxf = idx.reshape(B, N * NEIGHBORS, 1)

    dh = jnp.arange(INNER, dtype=jnp.int32) // DIM_HEAD
    hsum = (dh[:, None] == jnp.arange(HEADS, dtype=jnp.int32)[None, :]).astype(f32)
    hexp = hsum.T
    dm = jnp.arange(INNER, dtype=jnp.int32) % DIM_HEAD
    inv_freq = 1.0 / (10000.0 ** (jnp.arange(0, ROT_DIM, 2, dtype=f32) / ROT_DIM))
    iffr16 = (100.0 * inv_freq)[None, :]
    emat = ((dm[None, :] < ROT_DIM)
            & ((dm[None, :] // 2) == jnp.arange(NFREQ, dtype=jnp.int32)[:, None])
            ).astype(f32)
    ez = jnp.zeros_like(emat)
    ec = jnp.concatenate([emat, ez], axis=1)
    es = jnp.concatenate([ez, emat], axis=1)
    e4 = jnp.concatenate([ec, ec, es, es], axis=0)
    pmask = (dm >= ROT_DIM).astype(f32)[None, :]
    whi = W_out.astype(jnp.bfloat16).astype(f32)
    w2 = jnp.concatenate([whi, W_out - whi], axis=0)

    out, coors_out = pl.pallas_call(
        _attn_body,
        grid=(B, N // NB),
        in_specs=[
            pl.BlockSpec((1, NB, INNER), lambda b, i: (b, i, 0)),
            pl.BlockSpec((1, N, KVC), lambda b, i: (b, 0, 0)),
            pl.BlockSpec((1, NB, 3), lambda b, i: (b, i, 0)),
            pl.BlockSpec((1, NBK, 1), lambda b, i: (b, i, 0)),
            pl.BlockSpec((INNER, HEADS), lambda b, i: (0, 0)),
            pl.BlockSpec((HEADS, INNER), lambda b, i: (0, 0)),
            pl.BlockSpec((1, NFREQ), lambda b, i: (0, 0)),
            pl.BlockSpec((4 * NFREQ, 2 * INNER), lambda b, i: (0, 0)),
            pl.BlockSpec((1, INNER), lambda b, i: (0, 0)),
            pl.BlockSpec((HEADS, M_DIM * 4), lambda b, i: (0, 0)),
            pl.BlockSpec((1, M_DIM * 4), lambda b, i: (0, 0)),
            pl.BlockSpec((M_DIM * 4, 1), lambda b, i: (0, 0)),
            pl.BlockSpec((1, 1), lambda b, i: (0, 0)),
            pl.BlockSpec((1, 1), lambda b, i: (0, 0)),
            pl.BlockSpec((2 * INNER, DIM), lambda b, i: (0, 0)),
            pl.BlockSpec((1, DIM), lambda b, i: (0, 0)),
        ],
        out_specs=[
            pl.BlockSpec((1, NB, DIM), lambda b, i: (b, i, 0)),
            pl.BlockSpec((1, NB, 3), lambda b, i: (b, i, 0)),
        ],
        out_shape=[
            jax.ShapeDtypeStruct((B, N, DIM), f32),
            jax.ShapeDtypeStruct((B, N, 3), f32),
        ],
    )(q, kvc, coors, idxf, hsum, hexp, iffr16, e4, pmask, W_c1,
      b_c1.reshape(1, -1), W_c2, (b_c2 + 0.0).reshape(1, 1),
      ln_b.reshape(1, 1), w2, b_out.reshape(1, -1))
    return out, coors_out


# speedup vs baseline: 6.0928x; 1.2185x over previous
import jax
import jax.numpy as jnp
from jax.experimental import pallas as pl

B, N, DIM = 2, 1024, 512
HEADS, DIM_HEAD, M_DIM, NEIGHBORS = 8, 64, 4, 32
INNER = HEADS * DIM_HEAD
SCALE = DIM_HEAD ** -0.5
ROT_DIM = DIM_HEAD // 2
NFREQ = ROT_DIM // 2
KVC = 2 * INNER + 256

MB = 256
RB = 256
NB = 16
NBK = NB * NEIGHBORS


def _proj_body(x_ref, c_ref, wq_ref, wkv_ref, q_ref, kvc_ref):
    x = x_ref[...]
    q_ref[...] = jnp.dot(x, wq_ref[...], preferred_element_type=jnp.float32)
    kvc_ref[:, :2 * INNER] = jnp.dot(x, wkv_ref[...],
                                     preferred_element_type=jnp.float32)
    c = c_ref[...]
    chi = c.astype(jnp.bfloat16).astype(jnp.float32)
    clo = c - chi
    z = jnp.zeros((MB, 125), jnp.float32)
    kvc_ref[:, 2 * INNER:] = jnp.concatenate([chi, z, clo, z], axis=1)


def _topk_body(crow_ref, ct_ref, idx_ref):
    cr = crow_ref[0]
    ca = ct_ref[0]
    dx = cr[:, 0:1] - ca[0:1, :]
    dy = cr[:, 1:2] - ca[1:2, :]
    dz = cr[:, 2:3] - ca[2:3, :]
    cur = dx * dx + dy * dy + dz * dz
    iotaf = jax.lax.broadcasted_iota(jnp.int32, (RB, N), 1).astype(jnp.float32)
    lane32 = jax.lax.broadcasted_iota(jnp.int32, (RB, NEIGHBORS), 1)
    acc = jnp.zeros((RB, NEIGHBORS), jnp.float32)
    big = jnp.float32(N)
    for t in range(NEIGHBORS):
        m = jnp.min(cur, axis=1, keepdims=True)
        cand = jnp.where(cur == m, iotaf, big)
        amin = jnp.min(cand, axis=1, keepdims=True)
        acc = jnp.where(lane32 == t, amin, acc)
        cur = jnp.where(iotaf == amin, jnp.inf, cur)
    idx_ref[...] = acc.astype(jnp.int32)[None]


def _attn_body(q_ref, kvc_ref, cr_ref, idx_ref, hsum_ref, hexp_ref,
               iffr_ref, emat_ref, pmask_ref, wc1_ref, bc1_ref, wc2_ref,
               bc2_ref, lnb_ref, wout_ref, bout_ref, out_ref, cout_ref):
    idxc = idx_ref[0]
    iota = jax.lax.broadcasted_iota(jnp.int32, (NBK, N), 1)
    onehot = (idxc == iota).astype(jnp.float32)
    sel = jnp.dot(onehot, kvc_ref[0], preferred_element_type=jnp.float32)
    k_sel = sel[:, :INNER]
    v_sel = sel[:, INNER:2 * INNER]
    c_sel = (sel[:, 2 * INNER:2 * INNER + 3]
             + sel[:, 2 * INNER + 128:2 * INNER + 131])
    cr = cr_ref[0]
    c_ctr = jnp.broadcast_to(cr[:, None, :], (NB, NEIGHBORS, 3)).reshape(NBK, 3)
    rel = c_ctr - c_sel
    norm = jnp.sqrt(jnp.sum(rel * rel, axis=1, keepdims=True) + 1e-12)

    norm_row = norm.reshape(1, NBK)
    th16t = iffr_ref[...] * norm_row
    c16 = jnp.cos(th16t)
    s16 = jnp.sin(th16t)
    c16h = c16.astype(jnp.bfloat16).astype(jnp.float32)
    s16h = s16.astype(jnp.bfloat16).astype(jnp.float32)
    csin_t = jnp.concatenate([c16h, c16 - c16h, s16h, s16 - s16h], axis=0)
    cs = jax.lax.dot_general(csin_t, emat_ref[...],
                             (((0,), (0,)), ((), ())),
                             preferred_element_type=jnp.float32)
    cth = cs[:, :INNER] + pmask_ref[...]
    sth = cs[:, INNER:]
    lane = jax.lax.broadcasted_iota(jnp.int32, (1, INNER), 1)
    even = (lane % 2) == 0

    def rot(x):
        rl = jnp.concatenate([x[:, 1:], x[:, :1]], axis=1)
        rr = jnp.concatenate([x[:, -1:], x[:, :-1]], axis=1)
        return jnp.where(even, -rl, rr)

    k_rot = k_sel * cth + rot(k_sel) * sth
    v_rot = v_sel * cth + rot(v_sel) * sth

    q = q_ref[0]
    q_rep = jnp.broadcast_to(q[:, None, :], (NB, NEIGHBORS, INNER)).reshape(NBK, INNER)
    qk2 = jnp.dot(q_rep * k_rot, hsum_ref[...],
                  preferred_element_type=jnp.float32) * SCALE

    h = jnp.dot(qk2, wc1_ref[...], preferred_element_type=jnp.float32) + bc1_ref[...]
    h = 0.5 * h * (1.0 + jax.lax.erf(h * (2.0 ** -0.5)))
    cw = jnp.dot(h, wc2_ref[...], preferred_element_type=jnp.float32) + bc2_ref[...]

    normed = rel / jnp.maximum(norm, 1e-8)
    reln = lnb_ref[0, 0] * normed
    wrel = cw * reln
    cout_ref[...] = jnp.sum(wrel.reshape(NB, NEIGHBORS, 3), axis=1)[None]

    qk3 = qk2.reshape(NB, NEIGHBORS, HEADS)
    mx = jnp.max(qk3, axis=1, keepdims=True)
    e = jnp.exp(qk3 - mx)
    attn = e / jnp.sum(e, axis=1, keepdims=True)
    aexp = jnp.dot(attn.reshape(NBK, HEADS), hexp_ref[...],
                   preferred_element_type=jnp.float32)
    osum = jnp.sum((aexp * v_rot).reshape(NB, NEIGHBORS, INNER), axis=1)
    osum2 = jnp.concatenate([osum, osum], axis=1)
    out_ref[...] = (jnp.dot(osum2, wout_ref[...],
                            preferred_element_type=jnp.float32)
                    + bout_ref[...])[None]


def kernel(feats, coors, W_qkv, W_out, b_out, W_c1, b_c1, W_c2, b_c2, ln_w, ln_b):
    f32 = jnp.float32
    x = feats.reshape(B * N, DIM)
    cflat = coors.reshape(B * N, 3)
    Wq = W_qkv[:, :INNER]
    Wkv = W_qkv[:, INNER:]
    q2, kvc = pl.pallas_call(
        _proj_body,
        grid=(B * N // MB,),
        in_specs=[
            pl.BlockSpec((MB, DIM), lambda i: (i, 0)),
            pl.BlockSpec((MB, 3), lambda i: (i, 0)),
            pl.BlockSpec((DIM, INNER), lambda i: (0, 0)),
            pl.BlockSpec((DIM, 2 * INNER), lambda i: (0, 0)),
        ],
        out_specs=[
            pl.BlockSpec((MB, INNER), lambda i: (i, 0)),
            pl.BlockSpec((MB, KVC), lambda i: (i, 0)),
        ],
        out_shape=[
            jax.ShapeDtypeStruct((B * N, INNER), f32),
            jax.ShapeDtypeStruct((B * N, KVC), f32),
        ],
    )(x, cflat, Wq, Wkv)
    q = q2.reshape(B, N, INNER)
    kvc = kvc.reshape(B, N, KVC)

    coorsT = jnp.transpose(coors, (0, 2, 1))
    idx = pl.pallas_call(
        _topk_body,
        grid=(B, N // RB),
        in_specs=[
            pl.BlockSpec((1, RB, 3), lambda b, r: (b, r, 0)),
            pl.BlockSpec((1, 3, N), lambda b, r: (b, 0, 0)),
        ],
        out_specs=pl.BlockSpec((1, RB, NEIGHBORS), lambda b, r: (b, r, 0)),
        out_shape=jax.ShapeDtypeStruct((B, N, NEIGHBORS), jnp.int32),
    )(coors, coorsT)
    idxf = idx.reshape(B, N * NEIGHBORS, 1)

    dh = jnp.arange(INNER, dtype=jnp.int32) // DIM_HEAD
    hsum = (dh[:, None] == jnp.arange(HEADS, dtype=jnp.int32)[None, :]).astype(f32)
    hexp = hsum.T
    dm = jnp.arange(INNER, dtype=jnp.int32) % DIM_HEAD
    inv_freq = 1.0 / (10000.0 ** (jnp.arange(0, ROT_DIM, 2, dtype=f32) / ROT_DIM))
    iffr16 = (100.0 * inv_freq)[:, None]
    emat = ((dm[None, :] < ROT_DIM)
            & ((dm[None, :] // 2) == jnp.arange(NFREQ, dtype=jnp.int32)[:, None])
            ).astype(f32)
    ez = jnp.zeros_like(emat)
    ec = jnp.concatenate([emat, ez], axis=1)
    es = jnp.concatenate([ez, emat], axis=1)
    e4 = jnp.concatenate([ec, ec, es, es], axis=0)
    pmask = (dm >= ROT_DIM).astype(f32)[None, :]
    whi = W_out.astype(jnp.bfloat16).astype(f32)
    w2 = jnp.concatenate([whi, W_out - whi], axis=0)

    out, coors_out = pl.pallas_call(
        _attn_body,
        grid=(B, N // NB),
        in_specs=[
            pl.BlockSpec((1, NB, INNER), lambda b, i: (b, i, 0)),
            pl.BlockSpec((1, N, KVC), lambda b, i: (b, 0, 0)),
            pl.BlockSpec((1, NB, 3), lambda b, i: (b, i, 0)),
            pl.BlockSpec((1, NBK, 1), lambda b, i: (b, i, 0)),
            pl.BlockSpec((INNER, HEADS), lambda b, i: (0, 0)),
            pl.BlockSpec((HEADS, INNER), lambda b, i: (0, 0)),
            pl.BlockSpec((NFREQ, 1), lambda b, i: (0, 0)),
            pl.BlockSpec((4 * NFREQ, 2 * INNER), lambda b, i: (0, 0)),
            pl.BlockSpec((1, INNER), lambda b, i: (0, 0)),
            pl.BlockSpec((HEADS, M_DIM * 4), lambda b, i: (0, 0)),
            pl.BlockSpec((1, M_DIM * 4), lambda b, i: (0, 0)),
            pl.BlockSpec((M_DIM * 4, 1), lambda b, i: (0, 0)),
            pl.BlockSpec((1, 1), lambda b, i: (0, 0)),
            pl.BlockSpec((1, 1), lambda b, i: (0, 0)),
            pl.BlockSpec((2 * INNER, DIM), lambda b, i: (0, 0)),
            pl.BlockSpec((1, DIM), lambda b, i: (0, 0)),
        ],
        out_specs=[
            pl.BlockSpec((1, NB, DIM), lambda b, i: (b, i, 0)),
            pl.BlockSpec((1, NB, 3), lambda b, i: (b, i, 0)),
        ],
        out_shape=[
            jax.ShapeDtypeStruct((B, N, DIM), f32),
            jax.ShapeDtypeStruct((B, N, 3), f32),
        ],
    )(q, kvc, coors, idxf, hsum, hexp, iffr16, e4, pmask, W_c1,
      b_c1.reshape(1, -1), W_c2, (b_c2 + 0.0).reshape(1, 1),
      ln_b.reshape(1, 1), w2, b_out.reshape(1, -1))
    return out, coors_out
